# baseline (speedup 1.0000x reference)
import jax
import jax.numpy as jnp
from jax.experimental import pallas as pl

LN_EPS = 1e-12


def _emb_kernel(tt_ref, am_ref, pm_ref, inp_ref, img_ref, pos_ref, row0_ref,
                diff_ref, g_ref, b2_ref, cls_ref, mi_ref, out_ref, mask_ref):
    b = pl.program_id(0)
    L = inp_ref.shape[1]
    B = am_ref.shape[0]

    @pl.when(b == 0)
    def _():
        mask_ref[...] = jnp.concatenate(
            [jnp.ones((B, 1, 1), jnp.int32), am_ref[...], pm_ref[...]], axis=2)

    ttf = tt_ref[b]
    emb = inp_ref[0] + pos_ref[...] + (row0_ref[...] + ttf * diff_ref[...])
    mu = jnp.mean(emb, axis=1, keepdims=True)
    d = emb - mu
    var = jnp.mean(d * d, axis=1, keepdims=True)
    out_ref[0, 0:1, :] = cls_ref[...]
    out_ref[0, 1:1 + L, :] = g_ref[...] * d * jax.lax.rsqrt(var + LN_EPS) + b2_ref[...]
    out_ref[0, 1 + L:, :] = img_ref[0] + mi_ref[...]


def kernel(input_ids, attention_mask, token_type_ids, pixel_values, pixel_mask,
           inputs_embeds, image_embeds, image_token_type_idx,
           text_pos_emb, text_tok_type_emb, ln_gamma, ln_beta,
           cls_token, modality_tok_type_emb):
    B, L, H = inputs_embeds.shape
    NIMG = image_embeds.shape[1]
    S = 1 + L + NIMG

    mi = jnp.take(modality_tok_type_emb, image_token_type_idx, axis=0).reshape(1, H)
    b2 = (ln_beta + modality_tok_type_emb[0]).reshape(1, H)
    row0 = text_tok_type_emb[0:1, :]
    diff = text_tok_type_emb[1:2, :] - row0
    ttf3 = token_type_ids.astype(jnp.float32).reshape(B, L, 1)
    am3 = attention_mask.reshape(B, 1, L)
    pm3 = pixel_mask.reshape(B, 1, NIMG)

    out, mask3 = pl.pallas_call(
        _emb_kernel,
        grid=(B,),
        in_specs=[
            pl.BlockSpec((B, L, 1), lambda b: (0, 0, 0)),
            pl.BlockSpec((B, 1, L), lambda b: (0, 0, 0)),
            pl.BlockSpec((B, 1, NIMG), lambda b: (0, 0, 0)),
            pl.BlockSpec((1, L, H), lambda b: (b, 0, 0)),
            pl.BlockSpec((1, NIMG, H), lambda b: (b, 0, 0)),
            pl.BlockSpec((L, H), lambda b: (0, 0)),
            pl.BlockSpec((1, H), lambda b: (0, 0)),
            pl.BlockSpec((1, H), lambda b: (0, 0)),
            pl.BlockSpec((1, H), lambda b: (0, 0)),
            pl.BlockSpec((1, H), lambda b: (0, 0)),
            pl.BlockSpec((1, H), lambda b: (0, 0)),
            pl.BlockSpec((1, H), lambda b: (0, 0)),
        ],
        out_specs=[
            pl.BlockSpec((1, S, H), lambda b: (b, 0, 0)),
            pl.BlockSpec((B, 1, S), lambda b: (0, 0, 0)),
        ],
        out_shape=[
            jax.ShapeDtypeStruct((B, S, H), jnp.float32),
            jax.ShapeDtypeStruct((B, 1, S), jnp.int32),
        ],
    )(ttf3, am3, pm3, inputs_embeds, image_embeds,
      text_pos_emb[:L], row0, diff,
      ln_gamma.reshape(1, H), b2,
      cls_token.reshape(1, H), mi)

    return out, mask3.reshape(B, S)


# speedup vs baseline: 1.0257x; 1.0257x over previous
import jax
import jax.numpy as jnp
from jax.experimental import pallas as pl

LN_EPS = 1e-12


def _emb_kernel(tt_ref, mk_ref, inp_ref, img_ref, pos_ref, rows_ref,
                out_ref, mask_ref):
    b = pl.program_id(0)
    L = inp_ref.shape[1]
    B = mk_ref.shape[0]

    @pl.when(b == 0)
    def _():
        mask_ref[...] = jnp.concatenate(
            [jnp.ones((B, 1, 1), jnp.int32), mk_ref[...]], axis=2)

    row0 = rows_ref[0:1, :]
    diff = rows_ref[1:2, :]
    g = rows_ref[2:3, :]
    b2 = rows_ref[3:4, :]
    ttf = tt_ref[b]
    emb = inp_ref[0] + pos_ref[...] + (row0 + ttf * diff)
    mu = jnp.mean(emb, axis=1, keepdims=True)
    d = emb - mu
    var = jnp.mean(d * d, axis=1, keepdims=True)
    out_ref[0, 0:1, :] = rows_ref[4:5, :]
    out_ref[0, 1:1 + L, :] = g * d * jax.lax.rsqrt(var + LN_EPS) + b2
    out_ref[0, 1 + L:, :] = img_ref[0] + rows_ref[5:6, :]


def kernel(input_ids, attention_mask, token_type_ids, pixel_values, pixel_mask,
           inputs_embeds, image_embeds, image_token_type_idx,
           text_pos_emb, text_tok_type_emb, ln_gamma, ln_beta,
           cls_token, modality_tok_type_emb):
    B, L, H = inputs_embeds.shape
    NIMG = image_embeds.shape[1]
    S = 1 + L + NIMG

    mi = jnp.take(modality_tok_type_emb, image_token_type_idx, axis=0).reshape(1, H)
    b2 = (ln_beta + modality_tok_type_emb[0]).reshape(1, H)
    row0 = text_tok_type_emb[0:1, :]
    diff = text_tok_type_emb[1:2, :] - row0
    rows8 = jnp.concatenate(
        [row0, diff, ln_gamma.reshape(1, H), b2,
         cls_token.reshape(1, H), mi, jnp.zeros((2, H), jnp.float32)], axis=0)
    ttf3 = token_type_ids.astype(jnp.float32).reshape(B, L, 1)
    mk3 = jnp.concatenate([attention_mask, pixel_mask], axis=1).reshape(B, 1, L + NIMG)

    out, mask3 = pl.pallas_call(
        _emb_kernel,
        grid=(B,),
        in_specs=[
            pl.BlockSpec((B, L, 1), lambda b: (0, 0, 0)),
            pl.BlockSpec((B, 1, L + NIMG), lambda b: (0, 0, 0)),
            pl.BlockSpec((1, L, H), lambda b: (b, 0, 0)),
            pl.BlockSpec((1, NIMG, H), lambda b: (b, 0, 0)),
            pl.BlockSpec((L, H), lambda b: (0, 0)),
            pl.BlockSpec((8, H), lambda b: (0, 0)),
        ],
        out_specs=[
            pl.BlockSpec((1, S, H), lambda b: (b, 0, 0)),
            pl.BlockSpec((B, 1, S), lambda b: (0, 0, 0)),
        ],
        out_shape=[
            jax.ShapeDtypeStruct((B, S, H), jnp.float32),
            jax.ShapeDtypeStruct((B, 1, S), jnp.int32),
        ],
    )(ttf3, mk3, inputs_embeds, image_embeds, text_pos_emb[:L], rows8)

    return out, mask3.reshape(B, S)


# speedup vs baseline: 1.0438x; 1.0176x over previous
import jax
import jax.numpy as jnp
from jax.experimental import pallas as pl

LN_EPS = 1e-12
_GB = 2


def _emb_kernel(tt_ref, mk_ref, inp_ref, img_ref, pos_ref, rows_ref,
                out_ref, mask_ref):
    b = pl.program_id(0)
    L = inp_ref.shape[1]
    B = mk_ref.shape[0]

    @pl.when(b == 0)
    def _():
        mask_ref[...] = jnp.concatenate(
            [jnp.ones((B, 1, 1), jnp.int32), mk_ref[...]], axis=2)

    row0 = rows_ref[0:1, :]
    diff = rows_ref[1:2, :]
    g = rows_ref[2:3, :]
    b2 = rows_ref[3:4, :]
    pos = pos_ref[...]
    for bb in range(_GB):
        ttf = tt_ref[_GB * b + bb]
        emb = inp_ref[bb] + pos + (row0 + ttf * diff)
        mu = jnp.mean(emb, axis=1, keepdims=True)
        d = emb - mu
        var = jnp.mean(d * d, axis=1, keepdims=True)
        out_ref[bb, 0:1, :] = rows_ref[4:5, :]
        out_ref[bb, 1:1 + L, :] = g * d * jax.lax.rsqrt(var + LN_EPS) + b2
        out_ref[bb, 1 + L:, :] = img_ref[bb] + rows_ref[5:6, :]


def kernel(input_ids, attention_mask, token_type_ids, pixel_values, pixel_mask,
           inputs_embeds, image_embeds, image_token_type_idx,
           text_pos_emb, text_tok_type_emb, ln_gamma, ln_beta,
           cls_token, modality_tok_type_emb):
    B, L, H = inputs_embeds.shape
    NIMG = image_embeds.shape[1]
    S = 1 + L + NIMG

    mi = jnp.take(modality_tok_type_emb, image_token_type_idx, axis=0).reshape(1, H)
    b2 = (ln_beta + modality_tok_type_emb[0]).reshape(1, H)
    row0 = text_tok_type_emb[0:1, :]
    diff = text_tok_type_emb[1:2, :] - row0
    rows8 = jnp.concatenate(
        [row0, diff, ln_gamma.reshape(1, H), b2,
         cls_token.reshape(1, H), mi, jnp.zeros((2, H), jnp.float32)], axis=0)
    ttf3 = token_type_ids.astype(jnp.float32).reshape(B, L, 1)
    mk3 = jnp.concatenate([attention_mask, pixel_mask], axis=1).reshape(B, 1, L + NIMG)

    out, mask3 = pl.pallas_call(
        _emb_kernel,
        grid=(B // _GB,),
        in_specs=[
            pl.BlockSpec((B, L, 1), lambda b: (0, 0, 0)),
            pl.BlockSpec((B, 1, L + NIMG), lambda b: (0, 0, 0)),
            pl.BlockSpec((_GB, L, H), lambda b: (b, 0, 0)),
            pl.BlockSpec((_GB, NIMG, H), lambda b: (b, 0, 0)),
            pl.BlockSpec((L, H), lambda b: (0, 0)),
            pl.BlockSpec((8, H), lambda b: (0, 0)),
        ],
        out_specs=[
            pl.BlockSpec((_GB, S, H), lambda b: (b, 0, 0)),
            pl.BlockSpec((B, 1, S), lambda b: (0, 0, 0)),
        ],
        out_shape=[
            jax.ShapeDtypeStruct((B, S, H), jnp.float32),
            jax.ShapeDtypeStruct((B, 1, S), jnp.int32),
        ],
    )(ttf3, mk3, inputs_embeds, image_embeds, text_pos_emb[:L], rows8)

    return out, mask3.reshape(B, S)


# speedup vs baseline: 1.0453x; 1.0014x over previous
import jax
import jax.numpy as jnp
from jax.experimental import pallas as pl
from jax.experimental.pallas import tpu as pltpu

LN_EPS = 1e-12
_GB = 2
_NSLOT = 4


def _emb_kernel(tt_ref, mk_ref, inp_ref, img_ref, pos_ref, rows_ref,
                out_ref, mask_ref, slab, sem):
    b = pl.program_id(0)
    nb = pl.num_programs(0)
    L = inp_ref.shape[1]
    B = mk_ref.shape[0]

    @pl.when(b == 0)
    def _():
        mask_ref[...] = jnp.concatenate(
            [jnp.ones((B, 1, 1), jnp.int32), mk_ref[...]], axis=2)

    def slab_copy(g, s):
        return pltpu.make_async_copy(slab.at[s], out_ref.at[g], sem.at[s])

    row0 = rows_ref[0:1, :]
    diff = rows_ref[1:2, :]
    g_ = rows_ref[2:3, :]
    b2 = rows_ref[3:4, :]
    pos = pos_ref[...]
    for bb in range(_GB):
        g = _GB * b + bb
        slot = jax.lax.rem(g, _NSLOT)

        @pl.when(b >= 2)
        def _():
            slab_copy(g - _NSLOT, slot).wait()

        ttf = tt_ref[g]
        emb = inp_ref[bb] + pos + (row0 + ttf * diff)
        mu = jnp.mean(emb, axis=1, keepdims=True)
        d = emb - mu
        var = jnp.mean(d * d, axis=1, keepdims=True)
        slab[slot, 0:1, :] = rows_ref[4:5, :]
        slab[slot, 1:1 + L, :] = g_ * d * jax.lax.rsqrt(var + LN_EPS) + b2
        slab[slot, 1 + L:, :] = img_ref[bb] + rows_ref[5:6, :]
        slab_copy(g, slot).start()

    @pl.when(b == nb - 1)
    def _():
        for k in range(_NSLOT):
            g = _GB * b + 1 - k
            slab_copy(g, jax.lax.rem(g, _NSLOT)).wait()


def kernel(input_ids, attention_mask, token_type_ids, pixel_values, pixel_mask,
           inputs_embeds, image_embeds, image_token_type_idx,
           text_pos_emb, text_tok_type_emb, ln_gamma, ln_beta,
           cls_token, modality_tok_type_emb):
    B, L, H = inputs_embeds.shape
    NIMG = image_embeds.shape[1]
    S = 1 + L + NIMG

    mi = jnp.take(modality_tok_type_emb, image_token_type_idx, axis=0).reshape(1, H)
    b2 = (ln_beta + modality_tok_type_emb[0]).reshape(1, H)
    row0 = text_tok_type_emb[0:1, :]
    diff = text_tok_type_emb[1:2, :] - row0
    rows8 = jnp.concatenate(
        [row0, diff, ln_gamma.reshape(1, H), b2,
         cls_token.reshape(1, H), mi, jnp.zeros((2, H), jnp.float32)], axis=0)
    ttf3 = token_type_ids.astype(jnp.float32).reshape(B, L, 1)
    mk3 = jnp.concatenate([attention_mask, pixel_mask], axis=1).reshape(B, 1, L + NIMG)

    out, mask3 = pl.pallas_call(
        _emb_kernel,
        grid=(B // _GB,),
        in_specs=[
            pl.BlockSpec((B, L, 1), lambda b: (0, 0, 0)),
            pl.BlockSpec((B, 1, L + NIMG), lambda b: (0, 0, 0)),
            pl.BlockSpec((_GB, L, H), lambda b: (b, 0, 0)),
            pl.BlockSpec((_GB, NIMG, H), lambda b: (b, 0, 0)),
            pl.BlockSpec((L, H), lambda b: (0, 0)),
            pl.BlockSpec((8, H), lambda b: (0, 0)),
        ],
        out_specs=[
            pl.BlockSpec(memory_space=pl.MemorySpace.ANY),
            pl.BlockSpec((B, 1, S), lambda b: (0, 0, 0)),
        ],
        out_shape=[
            jax.ShapeDtypeStruct((B, S, H), jnp.float32),
            jax.ShapeDtypeStruct((B, 1, S), jnp.int32),
        ],
        scratch_shapes=[
            pltpu.VMEM((_NSLOT, S, H), jnp.float32),
            pltpu.SemaphoreType.DMA((_NSLOT,)),
        ],
    )(ttf3, mk3, inputs_embeds, image_embeds, text_pos_emb[:L], rows8)

    return out, mask3.reshape(B, S)


# speedup vs baseline: 1.0670x; 1.0208x over previous
import jax
import jax.numpy as jnp
from jax.experimental import pallas as pl
from jax.experimental.pallas import tpu as pltpu

LN_EPS = 1e-12
_GB = 4
_NSLOT = 4


def _emb_kernel(tt_ref, mk_ref, inp_ref, img_ref, pos_ref, rows_ref,
                out_ref, mask_ref, slab, sem):
    b = pl.program_id(0)
    nb = pl.num_programs(0)
    L = inp_ref.shape[1]
    B = mk_ref.shape[0]

    @pl.when(b == 0)
    def _():
        mask_ref[...] = jnp.concatenate(
            [jnp.ones((B, 1, 1), jnp.int32), mk_ref[...]], axis=2)

    def slab_copy(g, s):
        return pltpu.make_async_copy(slab.at[s], out_ref.at[g], sem.at[s])

    row0 = rows_ref[0:1, :]
    diff = rows_ref[1:2, :]
    g_ = rows_ref[2:3, :]
    b2 = rows_ref[3:4, :]
    pos = pos_ref[...]
    for bb in range(_GB):
        g = _GB * b + bb
        slot = jax.lax.rem(g, _NSLOT)

        @pl.when(b >= 1)
        def _():
            slab_copy(g - _NSLOT, slot).wait()

        ttf = tt_ref[g]
        emb = inp_ref[bb] + pos + (row0 + ttf * diff)
        mu = jnp.mean(emb, axis=1, keepdims=True)
        d = emb - mu
        var = jnp.mean(d * d, axis=1, keepdims=True)
        slab[slot, 0:1, :] = rows_ref[4:5, :]
        slab[slot, 1:1 + L, :] = g_ * d * jax.lax.rsqrt(var + LN_EPS) + b2
        slab[slot, 1 + L:, :] = img_ref[bb] + rows_ref[5:6, :]
        slab_copy(g, slot).start()

    @pl.when(b == nb - 1)
    def _():
        for k in range(_NSLOT):
            g = _GB * b + 1 - k
            slab_copy(g, jax.lax.rem(g, _NSLOT)).wait()


def kernel(input_ids, attention_mask, token_type_ids, pixel_values, pixel_mask,
           inputs_embeds, image_embeds, image_token_type_idx,
           text_pos_emb, text_tok_type_emb, ln_gamma, ln_beta,
           cls_token, modality_tok_type_emb):
    B, L, H = inputs_embeds.shape
    NIMG = image_embeds.shape[1]
    S = 1 + L + NIMG

    mi = jnp.take(modality_tok_type_emb, image_token_type_idx, axis=0).reshape(1, H)
    b2 = (ln_beta + modality_tok_type_emb[0]).reshape(1, H)
    row0 = text_tok_type_emb[0:1, :]
    diff = text_tok_type_emb[1:2, :] - row0
    rows8 = jnp.concatenate(
        [row0, diff, ln_gamma.reshape(1, H), b2,
         cls_token.reshape(1, H), mi, jnp.zeros((2, H), jnp.float32)], axis=0)
    ttf3 = token_type_ids.astype(jnp.float32).reshape(B, L, 1)
    mk3 = jnp.concatenate([attention_mask, pixel_mask], axis=1).reshape(B, 1, L + NIMG)

    out, mask3 = pl.pallas_call(
        _emb_kernel,
        grid=(B // _GB,),
        in_specs=[
            pl.BlockSpec((B, L, 1), lambda b: (0, 0, 0)),
            pl.BlockSpec((B, 1, L + NIMG), lambda b: (0, 0, 0)),
            pl.BlockSpec((_GB, L, H), lambda b: (b, 0, 0)),
            pl.BlockSpec((_GB, NIMG, H), lambda b: (b, 0, 0)),
            pl.BlockSpec((L, H), lambda b: (0, 0)),
            pl.BlockSpec((8, H), lambda b: (0, 0)),
        ],
        out_specs=[
            pl.BlockSpec(memory_space=pl.MemorySpace.ANY),
            pl.BlockSpec((B, 1, S), lambda b: (0, 0, 0)),
        ],
        out_shape=[
            jax.ShapeDtypeStruct((B, S, H), jnp.float32),
            jax.ShapeDtypeStruct((B, 1, S), jnp.int32),
        ],
        scratch_shapes=[
            pltpu.VMEM((_NSLOT, S, H), jnp.float32),
            pltpu.SemaphoreType.DMA((_NSLOT,)),
        ],
    )(ttf3, mk3, inputs_embeds, image_embeds, text_pos_emb[:L], rows8)

    return out, mask3.reshape(B, S)


# speedup vs baseline: 1.0723x; 1.0050x over previous
import jax
import jax.numpy as jnp
from jax.experimental import pallas as pl
from jax.experimental.pallas import tpu as pltpu

LN_EPS = 1e-12
_NSLOT = 4


def _emb_kernel(tt_ref, mk_ref, pos_ref, rows_ref, inp_hbm, img_hbm,
                out_hbm, mask_ref, ibuf, gbuf, slab, isem, gsem, osem):
    B, L, H = inp_hbm.shape
    NIMG = img_hbm.shape[1]

    mask_ref[...] = jnp.concatenate(
        [jnp.ones((B, 1, 1), jnp.int32), mk_ref[...]], axis=2)

    def in_copy(g, s):
        return pltpu.make_async_copy(inp_hbm.at[g], ibuf.at[s], isem.at[s])

    def im_copy(g, s):
        return pltpu.make_async_copy(img_hbm.at[g], gbuf.at[s], gsem.at[s])

    def out_copy(g, s):
        return pltpu.make_async_copy(slab.at[s], out_hbm.at[g], osem.at[s])

    for g in range(_NSLOT):
        in_copy(g, g).start()
        im_copy(g, g).start()

    row0 = rows_ref[0:1, :]
    diff = rows_ref[1:2, :]
    gam = rows_ref[2:3, :]
    b2 = rows_ref[3:4, :]
    pos = pos_ref[...]

    def step(g, _):
        s = jax.lax.rem(g, _NSLOT)
        in_copy(g, s).wait()
        im_copy(g, s).wait()

        @pl.when(g >= _NSLOT)
        def _():
            out_copy(g - _NSLOT, s).wait()

        ttf = tt_ref[g]
        emb = ibuf[s] + pos + (row0 + ttf * diff)
        mu = jnp.mean(emb, axis=1, keepdims=True)
        d = emb - mu
        var = jnp.mean(d * d, axis=1, keepdims=True)
        slab[s, 0:1, :] = rows_ref[4:5, :]
        slab[s, 1:1 + L, :] = gam * d * jax.lax.rsqrt(var + LN_EPS) + b2
        slab[s, 1 + L:, :] = gbuf[s] + rows_ref[5:6, :]
        out_copy(g, s).start()

        @pl.when(g + _NSLOT < B)
        def _():
            in_copy(g + _NSLOT, s).start()
            im_copy(g + _NSLOT, s).start()

        return 0

    jax.lax.fori_loop(0, B, step, 0)

    for g in range(B - _NSLOT, B):
        out_copy(g, g % _NSLOT).wait()


def kernel(input_ids, attention_mask, token_type_ids, pixel_values, pixel_mask,
           inputs_embeds, image_embeds, image_token_type_idx,
           text_pos_emb, text_tok_type_emb, ln_gamma, ln_beta,
           cls_token, modality_tok_type_emb):
    B, L, H = inputs_embeds.shape
    NIMG = image_embeds.shape[1]
    S = 1 + L + NIMG

    mi = jnp.take(modality_tok_type_emb, image_token_type_idx, axis=0).reshape(1, H)
    b2 = (ln_beta + modality_tok_type_emb[0]).reshape(1, H)
    row0 = text_tok_type_emb[0:1, :]
    diff = text_tok_type_emb[1:2, :] - row0
    rows8 = jnp.concatenate(
        [row0, diff, ln_gamma.reshape(1, H), b2,
         cls_token.reshape(1, H), mi, jnp.zeros((2, H), jnp.float32)], axis=0)
    ttf3 = token_type_ids.astype(jnp.float32).reshape(B, L, 1)
    mk3 = jnp.concatenate([attention_mask, pixel_mask], axis=1).reshape(B, 1, L + NIMG)

    out, mask3 = pl.pallas_call(
        _emb_kernel,
        grid=(1,),
        in_specs=[
            pl.BlockSpec((B, L, 1), lambda b: (0, 0, 0)),
            pl.BlockSpec((B, 1, L + NIMG), lambda b: (0, 0, 0)),
            pl.BlockSpec((L, H), lambda b: (0, 0)),
            pl.BlockSpec((8, H), lambda b: (0, 0)),
            pl.BlockSpec(memory_space=pl.MemorySpace.ANY),
            pl.BlockSpec(memory_space=pl.MemorySpace.ANY),
        ],
        out_specs=[
            pl.BlockSpec(memory_space=pl.MemorySpace.ANY),
            pl.BlockSpec((B, 1, S), lambda b: (0, 0, 0)),
        ],
        out_shape=[
            jax.ShapeDtypeStruct((B, S, H), jnp.float32),
            jax.ShapeDtypeStruct((B, 1, S), jnp.int32),
        ],
        scratch_shapes=[
            pltpu.VMEM((_NSLOT, L, H), jnp.float32),
            pltpu.VMEM((_NSLOT, NIMG, H), jnp.float32),
            pltpu.VMEM((_NSLOT, S, H), jnp.float32),
            pltpu.SemaphoreType.DMA((_NSLOT,)),
            pltpu.SemaphoreType.DMA((_NSLOT,)),
            pltpu.SemaphoreType.DMA((_NSLOT,)),
        ],
    )(ttf3, mk3, text_pos_emb[:L], rows8, inputs_embeds, image_embeds)

    return out, mask3.reshape(B, S)
